# baseline (speedup 1.0000x reference)
import functools
import math
import numpy as np
import jax
import jax.numpy as jnp
from jax import lax
from jax.experimental import pallas as pl
from jax.experimental.pallas import tpu as pltpu
from jax.experimental.pallas import tpu_sc as plsc

_D_MODEL = 768
_SEQ = 2048
_HALF = _SEQ // 2
_D_MEM = 256
_NHEAD = 8
_DH = _D_MEM // _NHEAD
_BATCH = 2
_F = _SEQ // 2 + 1
_FP = 1152
_K = max(1, min(int(25 * math.log(_F)), _F))
_BISECT_ITERS = 28
_TSEQ = 256


def _build_dft_consts():
    bf = jnp.bfloat16
    t = np.arange(_HALF, dtype=np.float64)
    f = np.arange(_F, dtype=np.float64)
    ang = 2.0 * np.pi * np.outer(f, t) / _SEQ
    ce = np.zeros((_FP, _HALF), np.float32)
    se = np.zeros((_FP, _HALF), np.float32)
    ce[:_F] = np.cos(ang)
    se[:_F] = -np.sin(ang)
    sgn = np.zeros((_FP, 1), np.float32)
    sgn[:_F, 0] = np.cos(np.pi * f)
    jrev = np.zeros((_HALF, _HALF), np.float32)
    jrev[np.arange(1, _HALF), _HALF - np.arange(1, _HALF)] = 1.0
    alpha = np.full((_F,), 2.0)
    alpha[0] = 1.0
    alpha[-1] = 1.0
    tt = np.arange(_FP, dtype=np.float64)
    ang2 = 2.0 * np.pi * np.outer(tt, f) / _SEQ
    pc = np.zeros((_FP, _FP), np.float32)
    ps = np.zeros((_FP, _FP), np.float32)
    pc[:, :_F] = np.cos(ang2) * (alpha / _SEQ)
    ps[:, :_F] = -np.sin(ang2) * (alpha / _SEQ)
    j2 = np.zeros((_HALF, _FP), np.float32)
    j2[np.arange(_HALF), _HALF - np.arange(_HALF)] = 1.0
    return (jnp.asarray(ce, bf), jnp.asarray(se, bf), jnp.asarray(sgn),
            jnp.asarray(jrev, bf), jnp.asarray(pc, bf), jnp.asarray(ps, bf),
            jnp.asarray(j2, bf))


_CE, _SE, _SGN, _JREV, _PC, _PS, _J2 = _build_dft_consts()


def _ln(x, g, b, eps=1e-5):
    mu = jnp.mean(x, axis=-1, keepdims=True)
    var = jnp.mean((x - mu) ** 2, axis=-1, keepdims=True)
    return (x - mu) * jax.lax.rsqrt(var + eps) * g + b


def _ctx_kernel(m_ref, wqkv_ref, bqkv_ref, wo_ref, bo_ref,
                w1_ref, b1_ref, w2_ref, b2_ref,
                g1_ref, be1_ref, g2_ref, be2_ref,
                pc_ref, aw_ref):
    m = m_ref[...]
    qkv = jax.lax.dot_general(m, wqkv_ref[...], (((1,), (1,)), ((), ())),
                              preferred_element_type=jnp.float32)
    qkv = qkv + bqkv_ref[...]
    q = qkv[:, :_D_MEM]
    k = qkv[:, _D_MEM:2 * _D_MEM]
    v = qkv[:, 2 * _D_MEM:]
    scale = 1.0 / math.sqrt(_DH)
    outs = []
    for h in range(_NHEAD):
        sl = slice(h * _DH, (h + 1) * _DH)
        qh, kh, vh = q[:, sl], k[:, sl], v[:, sl]
        s = jax.lax.dot_general(qh, kh, (((1,), (1,)), ((), ())),
                                preferred_element_type=jnp.float32) * scale
        p = jnp.exp(s)
        p = p * (1.0 / jnp.sum(p, axis=1, keepdims=True))
        outs.append(jnp.dot(p, vh, preferred_element_type=jnp.float32))
    o = jnp.concatenate(outs, axis=1)
    attn = jax.lax.dot_general(o, wo_ref[...], (((1,), (1,)), ((), ())),
                               preferred_element_type=jnp.float32) + bo_ref[...]
    m1 = _ln(m + attn, g1_ref[...], be1_ref[...])
    h1 = jax.lax.dot_general(m1, w1_ref[...], (((1,), (1,)), ((), ())),
                             preferred_element_type=jnp.float32) + b1_ref[...]
    h1 = h1 * 0.5 * (1.0 + jax.lax.erf(h1 / math.sqrt(2.0)))
    h2 = jax.lax.dot_general(h1, w2_ref[...], (((1,), (1,)), ((), ())),
                             preferred_element_type=jnp.float32) + b2_ref[...]
    m2 = _ln(m1 + h2, g2_ref[...], be2_ref[...])
    logits = jax.lax.dot_general(m2, m2, (((1,), (1,)), ((), ())),
                                 preferred_element_type=jnp.float32)
    logits = logits * (1.0 / math.sqrt(_D_MEM))
    pc_ref[...] = logits
    rowmax = jnp.max(logits, axis=1, keepdims=True)
    rowmin = jnp.min(logits, axis=1, keepdims=True)
    kf = jnp.float32(_K)

    def body(_, carry):
        lo, hi = carry
        mid = 0.5 * (lo + hi)
        cnt = jnp.sum(jnp.where(logits >= mid, 1.0, 0.0), axis=1, keepdims=True)
        take = cnt >= kf
        return jnp.where(take, mid, lo), jnp.where(take, hi, mid)

    lo, _ = jax.lax.fori_loop(0, _BISECT_ITERS, body, (rowmin, rowmax))
    sel = logits >= lo
    e = jnp.where(sel, jnp.exp(logits), 0.0)
    aw = (e / jnp.sum(e, axis=1, keepdims=True)).astype(jnp.bfloat16)
    aw_ref[...] = jnp.pad(aw, ((0, _FP - _F), (0, _FP - _F)))


_SC_NW = 32
_SC_CHUNK = 32848
_SC_TOT = _SC_NW * _SC_CHUNK


@functools.partial(
    pl.kernel,
    mesh=plsc.VectorSubcoreMesh(core_axis_name="c", subcore_axis_name="s"),
    out_type=jax.ShapeDtypeStruct((_SC_TOT,), jnp.float32),
    scratch_types=[pltpu.VMEM((_SC_CHUNK,), jnp.float32),
                   pltpu.VMEM((_SC_CHUNK,), jnp.float32)],
)
def _sc_sigmoid(lg_hbm, out_hbm, in_v, out_v):
    wid = lax.axis_index("s") * 2 + lax.axis_index("c")
    base = wid * _SC_CHUNK
    pltpu.sync_copy(lg_hbm.at[pl.ds(base, _SC_CHUNK)], in_v)

    def body(j, carry):
        xv = in_v[pl.ds(j * 16, 16)]
        sv = jnp.exp(-jnp.abs(xv))
        rv = 1.0 / (1.0 + sv)
        out_v[pl.ds(j * 16, 16)] = jnp.where(xv >= 0.0, rv, sv * rv)
        return carry

    lax.fori_loop(0, _SC_CHUNK // 16, body, 0)
    pltpu.sync_copy(out_v, out_hbm.at[pl.ds(base, _SC_CHUNK)])


def _mix_kernel(x_ref, ce_ref, se_ref, sgn_ref, jrev_ref, c_ref,
                pc_ref, ps_ref, j2_ref, z_ref):
    x = x_ref[0]
    x1 = x[:_HALF]
    x2 = x[_HALF:].astype(jnp.bfloat16)
    xsh = jnp.dot(jrev_ref[...], x2, preferred_element_type=jnp.float32)
    xe = (x1 + xsh).astype(jnp.bfloat16)
    xo = (x1 - xsh).astype(jnp.bfloat16)
    u_re = (jnp.dot(ce_ref[...], xe, preferred_element_type=jnp.float32)
            + sgn_ref[...] * x[_HALF:_HALF + 1])
    u_im = jnp.dot(se_ref[...], xo, preferred_element_type=jnp.float32)
    c = c_ref[...]
    v_re = jnp.dot(c, u_re.astype(jnp.bfloat16),
                   preferred_element_type=jnp.float32).astype(jnp.bfloat16)
    v_im = jnp.dot(c, u_im.astype(jnp.bfloat16),
                   preferred_element_type=jnp.float32).astype(jnp.bfloat16)
    cc = jnp.dot(pc_ref[...], v_re, preferred_element_type=jnp.float32)
    ss = jnp.dot(ps_ref[...], v_im, preferred_element_type=jnp.float32)
    y1 = (cc + ss)[:_HALF]
    w = (cc - ss).astype(jnp.bfloat16)
    y2 = jnp.dot(j2_ref[...], w, preferred_element_type=jnp.float32)
    y = jnp.concatenate([y1, y2], axis=0)
    z = x + y
    mu = jnp.mean(z, axis=-1, keepdims=True)
    var = jnp.mean((z - mu) ** 2, axis=-1, keepdims=True)
    z_ref[0] = (z - mu) * jax.lax.rsqrt(var + 1e-5)


def kernel(x_ts, M_frq, in_proj_w, in_proj_b, out_proj_w, out_proj_b,
           lin1_w, lin1_b, lin2_w, lin2_b,
           norm1_g, norm1_b, norm2_g, norm2_b):
    row = lambda a: a.reshape(1, -1)
    logits2d, c_pad = pl.pallas_call(
        _ctx_kernel,
        out_shape=[
            jax.ShapeDtypeStruct((_F, _F), jnp.float32),
            jax.ShapeDtypeStruct((_FP, _FP), jnp.bfloat16),
        ],
    )(M_frq, in_proj_w, row(in_proj_b), out_proj_w, row(out_proj_b),
      lin1_w, row(lin1_b), lin2_w, row(lin2_b),
      row(norm1_g), row(norm1_b), row(norm2_g), row(norm2_b))

    lg_flat = jnp.pad(logits2d.reshape(-1), (0, _SC_TOT - _F * _F))
    pc_flat = _sc_sigmoid(lg_flat)
    p_connect = pc_flat[:_F * _F].reshape(_F, _F)

    z = pl.pallas_call(
        _mix_kernel,
        grid=(_BATCH,),
        in_specs=[
            pl.BlockSpec((1, _SEQ, _D_MODEL), lambda b: (b, 0, 0)),
            pl.BlockSpec((_FP, _HALF), lambda b: (0, 0)),
            pl.BlockSpec((_FP, _HALF), lambda b: (0, 0)),
            pl.BlockSpec((_FP, 1), lambda b: (0, 0)),
            pl.BlockSpec((_HALF, _HALF), lambda b: (0, 0)),
            pl.BlockSpec((_FP, _FP), lambda b: (0, 0)),
            pl.BlockSpec((_FP, _FP), lambda b: (0, 0)),
            pl.BlockSpec((_FP, _FP), lambda b: (0, 0)),
            pl.BlockSpec((_HALF, _FP), lambda b: (0, 0)),
        ],
        out_specs=pl.BlockSpec((1, _SEQ, _D_MODEL), lambda b: (b, 0, 0)),
        out_shape=jax.ShapeDtypeStruct((_BATCH, _SEQ, _D_MODEL), jnp.float32),
    )(x_ts, _CE, _SE, _SGN, _JREV, c_pad, _PC, _PS, _J2)

    return (z, p_connect)


# speedup vs baseline: 1.4071x; 1.4071x over previous
import math
import numpy as np
import jax
import jax.numpy as jnp
from jax.experimental import pallas as pl

_D_MODEL = 768
_SEQ = 2048
_HALF = _SEQ // 2
_D_MEM = 256
_NHEAD = 8
_DH = _D_MEM // _NHEAD
_BATCH = 2
_F = _SEQ // 2 + 1
_FP = 1152
_K = max(1, min(int(25 * math.log(_F)), _F))
_BISECT_ITERS = 28
_TSEQ = 256


def _build_dft_consts():
    bf = jnp.bfloat16
    t = np.arange(_HALF, dtype=np.float64)
    f = np.arange(_F, dtype=np.float64)
    ang = 2.0 * np.pi * np.outer(f, t) / _SEQ
    ce = np.zeros((_FP, _HALF), np.float32)
    se = np.zeros((_FP, _HALF), np.float32)
    ce[:_F] = np.cos(ang)
    se[:_F] = -np.sin(ang)
    sgn = np.zeros((_FP, 1), np.float32)
    sgn[:_F, 0] = np.cos(np.pi * f)
    jrev = np.zeros((_HALF, _HALF), np.float32)
    jrev[np.arange(1, _HALF), _HALF - np.arange(1, _HALF)] = 1.0
    alpha = np.full((_F,), 2.0)
    alpha[0] = 1.0
    alpha[-1] = 1.0
    tt = np.arange(_FP, dtype=np.float64)
    ang2 = 2.0 * np.pi * np.outer(tt, f) / _SEQ
    pc = np.zeros((_FP, _FP), np.float32)
    ps = np.zeros((_FP, _FP), np.float32)
    pc[:, :_F] = np.cos(ang2) * (alpha / _SEQ)
    ps[:, :_F] = -np.sin(ang2) * (alpha / _SEQ)
    j2 = np.zeros((_HALF, _FP), np.float32)
    j2[np.arange(_HALF), _HALF - np.arange(_HALF)] = 1.0
    return (jnp.asarray(ce, bf), jnp.asarray(se, bf), jnp.asarray(sgn),
            jnp.asarray(jrev, bf), jnp.asarray(pc, bf), jnp.asarray(ps, bf),
            jnp.asarray(j2, bf))


_CE, _SE, _SGN, _JREV, _PC, _PS, _J2 = _build_dft_consts()


def _ln(x, g, b, eps=1e-5):
    mu = jnp.mean(x, axis=-1, keepdims=True)
    var = jnp.mean((x - mu) ** 2, axis=-1, keepdims=True)
    return (x - mu) * jax.lax.rsqrt(var + eps) * g + b


def _ctx_kernel(m_ref, wqkv_ref, bqkv_ref, wo_ref, bo_ref,
                w1_ref, b1_ref, w2_ref, b2_ref,
                g1_ref, be1_ref, g2_ref, be2_ref,
                pc_ref, aw_ref):
    m = m_ref[...]
    qkv = jax.lax.dot_general(m, wqkv_ref[...], (((1,), (1,)), ((), ())),
                              preferred_element_type=jnp.float32)
    qkv = qkv + bqkv_ref[...]
    q = qkv[:, :_D_MEM]
    k = qkv[:, _D_MEM:2 * _D_MEM]
    v = qkv[:, 2 * _D_MEM:]
    scale = 1.0 / math.sqrt(_DH)
    outs = []
    for h in range(_NHEAD):
        sl = slice(h * _DH, (h + 1) * _DH)
        qh, kh, vh = q[:, sl], k[:, sl], v[:, sl]
        s = jax.lax.dot_general(qh, kh, (((1,), (1,)), ((), ())),
                                preferred_element_type=jnp.float32) * scale
        p = jnp.exp(s)
        p = p * (1.0 / jnp.sum(p, axis=1, keepdims=True))
        outs.append(jnp.dot(p, vh, preferred_element_type=jnp.float32))
    o = jnp.concatenate(outs, axis=1)
    attn = jax.lax.dot_general(o, wo_ref[...], (((1,), (1,)), ((), ())),
                               preferred_element_type=jnp.float32) + bo_ref[...]
    m1 = _ln(m + attn, g1_ref[...], be1_ref[...])
    h1 = jax.lax.dot_general(m1, w1_ref[...], (((1,), (1,)), ((), ())),
                             preferred_element_type=jnp.float32) + b1_ref[...]
    h1 = h1 * 0.5 * (1.0 + jax.lax.erf(h1 / math.sqrt(2.0)))
    h2 = jax.lax.dot_general(h1, w2_ref[...], (((1,), (1,)), ((), ())),
                             preferred_element_type=jnp.float32) + b2_ref[...]
    m2 = _ln(m1 + h2, g2_ref[...], be2_ref[...])
    logits = jax.lax.dot_general(m2, m2, (((1,), (1,)), ((), ())),
                                 preferred_element_type=jnp.float32)
    logits = logits * (1.0 / math.sqrt(_D_MEM))
    s = jnp.exp(-jnp.abs(logits))
    r = 1.0 / (1.0 + s)
    pc_ref[...] = jnp.where(logits >= 0.0, r, s * r)
    rowmax = jnp.max(logits, axis=1, keepdims=True)
    rowmin = jnp.min(logits, axis=1, keepdims=True)
    kf = jnp.float32(_K)

    def body(_, carry):
        lo, hi = carry
        mid = 0.5 * (lo + hi)
        cnt = jnp.sum(jnp.where(logits >= mid, 1.0, 0.0), axis=1, keepdims=True)
        take = cnt >= kf
        return jnp.where(take, mid, lo), jnp.where(take, hi, mid)

    lo, _ = jax.lax.fori_loop(0, _BISECT_ITERS, body, (rowmin, rowmax))
    sel = logits >= lo
    e = jnp.where(sel, jnp.exp(logits), 0.0)
    aw = (e / jnp.sum(e, axis=1, keepdims=True)).astype(jnp.bfloat16)
    aw_ref[...] = jnp.pad(aw, ((0, _FP - _F), (0, _FP - _F)))


def _mix_kernel(x_ref, ce_ref, se_ref, sgn_ref, jrev_ref, c_ref,
                pc_ref, ps_ref, j2_ref, z_ref):
    x = x_ref[0]
    x1 = x[:_HALF]
    x2 = x[_HALF:].astype(jnp.bfloat16)
    xsh = jnp.dot(jrev_ref[...], x2, preferred_element_type=jnp.float32)
    xe = (x1 + xsh).astype(jnp.bfloat16)
    xo = (x1 - xsh).astype(jnp.bfloat16)
    u_re = (jnp.dot(ce_ref[...], xe, preferred_element_type=jnp.float32)
            + sgn_ref[...] * x[_HALF:_HALF + 1])
    u_im = jnp.dot(se_ref[...], xo, preferred_element_type=jnp.float32)
    c = c_ref[...]
    v_re = jnp.dot(c, u_re.astype(jnp.bfloat16),
                   preferred_element_type=jnp.float32).astype(jnp.bfloat16)
    v_im = jnp.dot(c, u_im.astype(jnp.bfloat16),
                   preferred_element_type=jnp.float32).astype(jnp.bfloat16)
    cc = jnp.dot(pc_ref[...], v_re, preferred_element_type=jnp.float32)
    ss = jnp.dot(ps_ref[...], v_im, preferred_element_type=jnp.float32)
    y1 = (cc + ss)[:_HALF]
    w = (cc - ss).astype(jnp.bfloat16)
    y2 = jnp.dot(j2_ref[...], w, preferred_element_type=jnp.float32)
    y = jnp.concatenate([y1, y2], axis=0)
    z = x + y
    mu = jnp.mean(z, axis=-1, keepdims=True)
    var = jnp.mean((z - mu) ** 2, axis=-1, keepdims=True)
    z_ref[0] = (z - mu) * jax.lax.rsqrt(var + 1e-5)


def kernel(x_ts, M_frq, in_proj_w, in_proj_b, out_proj_w, out_proj_b,
           lin1_w, lin1_b, lin2_w, lin2_b,
           norm1_g, norm1_b, norm2_g, norm2_b):
    row = lambda a: a.reshape(1, -1)
    p_connect, c_pad = pl.pallas_call(
        _ctx_kernel,
        out_shape=[
            jax.ShapeDtypeStruct((_F, _F), jnp.float32),
            jax.ShapeDtypeStruct((_FP, _FP), jnp.bfloat16),
        ],
    )(M_frq, in_proj_w, row(in_proj_b), out_proj_w, row(out_proj_b),
      lin1_w, row(lin1_b), lin2_w, row(lin2_b),
      row(norm1_g), row(norm1_b), row(norm2_g), row(norm2_b))

    z = pl.pallas_call(
        _mix_kernel,
        grid=(_BATCH,),
        in_specs=[
            pl.BlockSpec((1, _SEQ, _D_MODEL), lambda b: (b, 0, 0)),
            pl.BlockSpec((_FP, _HALF), lambda b: (0, 0)),
            pl.BlockSpec((_FP, _HALF), lambda b: (0, 0)),
            pl.BlockSpec((_FP, 1), lambda b: (0, 0)),
            pl.BlockSpec((_HALF, _HALF), lambda b: (0, 0)),
            pl.BlockSpec((_FP, _FP), lambda b: (0, 0)),
            pl.BlockSpec((_FP, _FP), lambda b: (0, 0)),
            pl.BlockSpec((_FP, _FP), lambda b: (0, 0)),
            pl.BlockSpec((_HALF, _FP), lambda b: (0, 0)),
        ],
        out_specs=pl.BlockSpec((1, _SEQ, _D_MODEL), lambda b: (b, 0, 0)),
        out_shape=jax.ShapeDtypeStruct((_BATCH, _SEQ, _D_MODEL), jnp.float32),
    )(x_ts, _CE, _SE, _SGN, _JREV, c_pad, _PC, _PS, _J2)

    return (z, p_connect)
